# baseline (speedup 1.0000x reference)
import functools

import jax
import jax.numpy as jnp
from jax import lax
from jax.experimental import pallas as pl
from jax.experimental.pallas import tpu as pltpu
from jax.experimental.pallas import tpu_sc as plsc

B = 16384
D = 64
OUT_D = 4 * D
NUM_CORES = 2
NUM_SUBCORES = 16
NW = NUM_CORES * NUM_SUBCORES
BPW = B // NW


@functools.partial(
    pl.kernel,
    mesh=plsc.VectorSubcoreMesh(core_axis_name="c", subcore_axis_name="s"),
    out_type=jax.ShapeDtypeStruct((B, OUT_D), jnp.float32),
    scratch_types=[
        pltpu.VMEM((BPW,), jnp.int32),
        pltpu.VMEM((BPW, D), jnp.float32),
        pltpu.SemaphoreType.DMA,
        pltpu.SemaphoreType.DMA,
    ],
    compiler_params=pltpu.CompilerParams(use_tc_tiling_on_sc=False),
)
def _gather_concat(z_src, z_dst, idx_hbm, t_enc, table, out, idx_v, rows_v,
                   gsem, csem):
    wid = lax.axis_index("s") * NUM_CORES + lax.axis_index("c")
    base = wid * BPW
    rows = pl.ds(base, BPW)
    pltpu.sync_copy(idx_hbm.at[rows], idx_v)
    gather = pltpu.async_copy(table.at[idx_v], rows_v, gsem)
    c1 = pltpu.async_copy(z_src.at[rows], out.at[rows, pl.ds(0, D)], csem)
    c2 = pltpu.async_copy(z_dst.at[rows], out.at[rows, pl.ds(D, D)], csem)
    c3 = pltpu.async_copy(t_enc.at[rows], out.at[rows, pl.ds(3 * D, D)], csem)
    gather.wait()
    pltpu.sync_copy(rows_v, out.at[rows, pl.ds(2 * D, D)])
    c1.wait()
    c2.wait()
    c3.wait()


def kernel(z_src, z_dst, raw_msg, t_enc, embedding_weight):
    idx = raw_msg.astype(jnp.int32)
    return _gather_concat(z_src, z_dst, idx, t_enc, embedding_weight)


# speedup vs baseline: 1.5593x; 1.5593x over previous
import functools

import jax
import jax.numpy as jnp
from jax import lax
from jax.experimental import pallas as pl
from jax.experimental.pallas import tpu as pltpu
from jax.experimental.pallas import tpu_sc as plsc

B = 16384
D = 64
OUT_D = 4 * D
NUM_CORES = 2
NUM_SUBCORES = 16
NW = NUM_CORES * NUM_SUBCORES
BPW = B // NW


@functools.partial(
    pl.kernel,
    mesh=plsc.VectorSubcoreMesh(core_axis_name="c", subcore_axis_name="s"),
    out_type=jax.ShapeDtypeStruct((B, D), jnp.float32),
    scratch_types=[
        pltpu.VMEM((BPW,), jnp.int32),
        pltpu.VMEM((BPW, D), jnp.float32),
        pltpu.SemaphoreType.DMA,
    ],
    compiler_params=pltpu.CompilerParams(use_tc_tiling_on_sc=False),
)
def _sc_gather(idx_hbm, table, out, idx_v, rows_v, sem):
    wid = lax.axis_index("s") * NUM_CORES + lax.axis_index("c")
    base = wid * BPW
    rows = pl.ds(base, BPW)
    pltpu.sync_copy(idx_hbm.at[rows], idx_v)
    pltpu.async_copy(table.at[idx_v], rows_v, sem).wait()
    pltpu.sync_copy(rows_v, out.at[rows])


def _concat_body(z_src_ref, z_dst_ref, pos_ref, t_ref, out_ref):
    out_ref[...] = jnp.concatenate(
        [z_src_ref[...], z_dst_ref[...], pos_ref[...], t_ref[...]], axis=-1)


_R = 2048
_concat = pl.pallas_call(
    _concat_body,
    grid=(B // _R,),
    in_specs=[pl.BlockSpec((_R, D), lambda i: (i, 0))] * 4,
    out_specs=pl.BlockSpec((_R, OUT_D), lambda i: (i, 0)),
    out_shape=jax.ShapeDtypeStruct((B, OUT_D), jnp.float32),
)


def kernel(z_src, z_dst, raw_msg, t_enc, embedding_weight):
    idx = raw_msg.astype(jnp.int32)
    pos_msg = _sc_gather(idx, embedding_weight)
    return _concat(z_src, z_dst, pos_msg, t_enc)


# speedup vs baseline: 3.3098x; 2.1226x over previous
import functools

import jax
import jax.numpy as jnp
from jax import lax
from jax.experimental import pallas as pl
from jax.experimental.pallas import tpu as pltpu
from jax.experimental.pallas import tpu_sc as plsc

B = 16384
D = 64
OUT_D = 4 * D
NUM_CORES = 2
NUM_SUBCORES = 16
NW = NUM_CORES * NUM_SUBCORES
BPW = B // NW
G = 16
NGROUP = BPW // G


@functools.partial(
    pl.kernel,
    mesh=plsc.VectorSubcoreMesh(core_axis_name="c", subcore_axis_name="s"),
    out_type=jax.ShapeDtypeStruct((B, D), jnp.float32),
    scratch_types=[
        pltpu.VMEM((BPW,), jnp.int32),
        pltpu.VMEM((G, 8, D), jnp.float32),
        pltpu.VMEM((BPW, D), jnp.float32),
        pltpu.SemaphoreType.DMA,
    ],
)
def _sc_gather(idx_hbm, table3, out, idx_v, tiles_v, rows_v, sem):
    wid = lax.axis_index("s") * NUM_CORES + lax.axis_index("c")
    base = wid * BPW
    pltpu.sync_copy(idx_hbm.at[pl.ds(base, BPW)], idx_v)

    def group_body(g, _):
        gbase = g * G
        vec = idx_v[pl.ds(gbase, G)]
        tvec = lax.shift_right_logical(vec, 3)
        rvec = vec & 7
        handles = []
        for j in range(G):
            handles.append(pltpu.async_copy(
                table3.at[pl.ds(tvec[j], 1)], tiles_v.at[pl.ds(j, 1)], sem))
        for h in handles:
            h.wait()
        for j in range(G):
            for k in range(D // 16):
                rows_v[gbase + j, pl.ds(k * 16, 16)] = (
                    tiles_v[j, rvec[j], pl.ds(k * 16, 16)])
        return 0

    lax.fori_loop(0, NGROUP, group_body, 0)
    pltpu.sync_copy(rows_v, out.at[pl.ds(base, BPW)])


def _concat_body(z_src_ref, z_dst_ref, pos_ref, t_ref, out_ref):
    out_ref[...] = jnp.concatenate(
        [z_src_ref[...], z_dst_ref[...], pos_ref[...], t_ref[...]], axis=-1)


_R = 2048
_concat = pl.pallas_call(
    _concat_body,
    grid=(B // _R,),
    in_specs=[pl.BlockSpec((_R, D), lambda i: (i, 0))] * 4,
    out_specs=pl.BlockSpec((_R, OUT_D), lambda i: (i, 0)),
    out_shape=jax.ShapeDtypeStruct((B, OUT_D), jnp.float32),
)


def kernel(z_src, z_dst, raw_msg, t_enc, embedding_weight):
    idx = raw_msg.astype(jnp.int32)
    table3 = embedding_weight.reshape(125000, 8, D)
    pos_msg = _sc_gather(idx, table3)
    return _concat(z_src, z_dst, pos_msg, t_enc)
